# baseline (speedup 1.0000x reference)
import functools

import jax
import jax.numpy as jnp
from jax import lax
from jax.experimental import pallas as pl
from jax.experimental.pallas import tpu as pltpu
from jax.experimental.pallas import tpu_sc as plsc

N = 1024
E = 16384
HEADS = 16
TABLE_V = 50
MAX_ITERS = 49

NC = 2
NS = 16
LANES = 16
NW = NC * NS

ROWS_PER_TILE = N // NW
CHUNK = 2048
ROWS_PER_CHUNK = CHUNK // N
CHUNKS = ROWS_PER_TILE // ROWS_PER_CHUNK

_mesh = plsc.VectorSubcoreMesh(core_axis_name="c", subcore_axis_name="s")
_sc_params = pltpu.CompilerParams(needs_layout_passes=False)


@functools.partial(
    pl.kernel,
    out_type=jax.ShapeDtypeStruct((N, N), jnp.float32),
    mesh=_mesh,
    scratch_types=[
        pltpu.VMEM((2, E), jnp.int32),
        pltpu.VMEM((ROWS_PER_TILE, N), jnp.float32),
    ],
    compiler_params=_sc_params,
)
def _build_adj(edge_hbm, a_hbm, edges_v, a_v):
    wid = lax.axis_index("s") * NC + lax.axis_index("c")
    base_row = wid * ROWS_PER_TILE
    pltpu.sync_copy(edge_hbm, edges_v)

    @plsc.parallel_loop(0, ROWS_PER_TILE * N // LANES, 1, unroll=4)
    def _(i):
        r = lax.shift_right_logical(i, 6)
        c = lax.shift_left(jnp.bitwise_and(i, 63), 4)
        a_v[r, pl.ds(c, LANES)] = jnp.zeros((LANES,), jnp.float32)

    ones = jnp.ones((LANES,), jnp.float32)

    @plsc.parallel_loop(0, E // LANES, 1, unroll=4)
    def _(i):
        src = edges_v[0, pl.ds(i * LANES, LANES)]
        dst = edges_v[1, pl.ds(i * LANES, LANES)]
        mask = (src >= base_row) & (src < base_row + ROWS_PER_TILE)
        local_r = jnp.where(mask, src - base_row, 0)
        plsc.store_scatter(a_v, [local_r, dst], ones, mask=mask)
    pltpu.sync_copy(a_v, a_hbm.at[pl.ds(base_row, ROWS_PER_TILE), :])


def _bfs_body(a_ref, dist_ref, abf_ref, frontier_ref, cnt_ref):
    abf_ref[...] = a_ref[...].astype(jnp.bfloat16)
    row = lax.broadcasted_iota(jnp.int32, (N, N), 0)
    col = lax.broadcasted_iota(jnp.int32, (N, N), 1)
    diag = row == col
    frontier_ref[...] = jnp.where(diag, 1.0, 0.0).astype(jnp.bfloat16)
    dist_ref[...] = jnp.zeros((N, N), jnp.int32)

    def body(i, done):
        @pl.when(jnp.logical_not(done))
        def _():
            nxt = lax.dot_general(
                frontier_ref[...], abf_ref[...],
                (((1,), (0,)), ((), ())),
                preferred_element_type=jnp.float32,
            )
            new = (nxt > 0.0) & (dist_ref[...] == 0) & jnp.logical_not(diag)
            dist_ref[...] = jnp.where(new, i + 1, dist_ref[...])
            frontier_ref[...] = new.astype(jnp.bfloat16)
            cnt_ref[0] = jnp.sum(new.astype(jnp.int32))

        return jnp.logical_or(done, cnt_ref[0] == 0)

    lax.fori_loop(0, MAX_ITERS, body, False)


_bfs = pl.pallas_call(
    _bfs_body,
    out_shape=jax.ShapeDtypeStruct((N, N), jnp.int32),
    scratch_shapes=[
        pltpu.VMEM((N, N), jnp.bfloat16),
        pltpu.VMEM((N, N), jnp.bfloat16),
        pltpu.SMEM((1,), jnp.int32),
    ],
)


TC_HEADS = 8
SC_HEADS = HEADS - TC_HEADS
TBL_PAD = 128
BLK = 128


def _tc_lookup_body(tbl_ref, dist_ref, out_ref):
    d = dist_ref[...]
    for h in range(TC_HEADS):
        tmat = jnp.broadcast_to(tbl_ref[h, :][None, :], (BLK, TBL_PAD))
        out_ref[h] = jnp.take_along_axis(tmat, d, axis=1)


_tc_lookup = pl.pallas_call(
    _tc_lookup_body,
    grid=(N // BLK,),
    in_specs=[
        pl.BlockSpec((TC_HEADS, TBL_PAD), lambda i: (0, 0)),
        pl.BlockSpec((BLK, N), lambda i: (i, 0)),
    ],
    out_specs=pl.BlockSpec((TC_HEADS, BLK, N), lambda i: (0, i, 0)),
    out_shape=jax.ShapeDtypeStruct((TC_HEADS, N, N), jnp.float32),
)


@functools.partial(
    pl.kernel,
    out_type=jax.ShapeDtypeStruct((SC_HEADS, N, N), jnp.float32),
    mesh=_mesh,
    scratch_types=[
        pltpu.VMEM((SC_HEADS * TABLE_V,), jnp.float32),
        pltpu.VMEM((2, ROWS_PER_CHUNK, N), jnp.int32),
        pltpu.VMEM((2, SC_HEADS, ROWS_PER_CHUNK, N), jnp.float32),
        pltpu.SemaphoreType.DMA((2,)),
        pltpu.SemaphoreType.DMA((2,)),
    ],
    compiler_params=_sc_params,
)
def _lookup(dist_hbm, ftab_hbm, out_hbm, tab_v, idx_v, out_v, in_sems, out_sems):
    wid = lax.axis_index("s") * NC + lax.axis_index("c")
    row_base = wid * ROWS_PER_TILE
    pltpu.sync_copy(ftab_hbm, tab_v)

    def in_desc(c, b):
        return pltpu.make_async_copy(
            dist_hbm.at[pl.ds(row_base + c * ROWS_PER_CHUNK, ROWS_PER_CHUNK), :],
            idx_v.at[b],
            in_sems.at[b],
        )

    def out_desc(c, b):
        return pltpu.make_async_copy(
            out_v.at[b],
            out_hbm.at[:, pl.ds(row_base + c * ROWS_PER_CHUNK, ROWS_PER_CHUNK), :],
            out_sems.at[b],
        )

    in_desc(0, 0).start()
    in_desc(1, 1).start()

    def chunk_pair(cp, carry):
        for b in range(2):
            c = cp * 2 + b
            in_desc(c, b).wait()

            @pl.when(cp > 0)
            def _():
                out_desc(c, b).wait()

            for r in range(ROWS_PER_CHUNK):
                @plsc.parallel_loop(0, N // LANES, 1, unroll=4)
                def _(cg):
                    col = cg * LANES
                    iv = idx_v[b, r, pl.ds(col, LANES)]
                    for h in range(SC_HEADS):
                        vals = plsc.load_gather(tab_v, [iv + h * TABLE_V])
                        out_v[b, h, r, pl.ds(col, LANES)] = vals

            out_desc(c, b).start()

            @pl.when(c + 2 < CHUNKS)
            def _():
                in_desc(c + 2, b).start()
        return carry

    lax.fori_loop(0, CHUNKS // 2, chunk_pair, 0)
    out_desc(CHUNKS - 2, 0).wait()
    out_desc(CHUNKS - 1, 1).wait()


def kernel(x, edge_idx, table):
    del x
    a = _build_adj(edge_idx)
    dist = _bfs(a)
    tab_t = jnp.transpose(table)
    tbl_pad = jnp.zeros((TC_HEADS, TBL_PAD), jnp.float32).at[:, :TABLE_V].set(
        tab_t[:TC_HEADS])
    ftab_sc = tab_t[TC_HEADS:].reshape(SC_HEADS * TABLE_V)
    out_tc = _tc_lookup(tbl_pad, dist)
    out_sc = _lookup(dist, ftab_sc)
    return jnp.concatenate([out_tc, out_sc], axis=0)


# speedup vs baseline: 1.4874x; 1.4874x over previous
import functools

import jax
import jax.numpy as jnp
from jax import lax
from jax.experimental import pallas as pl
from jax.experimental.pallas import tpu as pltpu
from jax.experimental.pallas import tpu_sc as plsc

N = 1024
E = 16384
HEADS = 16
TABLE_V = 50
MAX_ITERS = 49

NC = 2
NS = 16
LANES = 16
NW = NC * NS

ROWS_PER_TILE = N // NW
CHUNK = 2048
ROWS_PER_CHUNK = CHUNK // N
CHUNKS = ROWS_PER_TILE // ROWS_PER_CHUNK

_mesh = plsc.VectorSubcoreMesh(core_axis_name="c", subcore_axis_name="s")
_sc_params = pltpu.CompilerParams(needs_layout_passes=False)


@functools.partial(
    pl.kernel,
    out_type=jax.ShapeDtypeStruct((N, N), jnp.float32),
    mesh=_mesh,
    scratch_types=[
        pltpu.VMEM((2, E), jnp.int32),
        pltpu.VMEM((ROWS_PER_TILE, N), jnp.float32),
    ],
    compiler_params=_sc_params,
)
def _build_adj(edge_hbm, a_hbm, edges_v, a_v):
    wid = lax.axis_index("s") * NC + lax.axis_index("c")
    base_row = wid * ROWS_PER_TILE
    pltpu.sync_copy(edge_hbm, edges_v)

    @plsc.parallel_loop(0, ROWS_PER_TILE * N // LANES, 1, unroll=4)
    def _(i):
        r = lax.shift_right_logical(i, 6)
        c = lax.shift_left(jnp.bitwise_and(i, 63), 4)
        a_v[r, pl.ds(c, LANES)] = jnp.zeros((LANES,), jnp.float32)

    ones = jnp.ones((LANES,), jnp.float32)

    @plsc.parallel_loop(0, E // LANES, 1, unroll=4)
    def _(i):
        src = edges_v[0, pl.ds(i * LANES, LANES)]
        dst = edges_v[1, pl.ds(i * LANES, LANES)]
        mask = (src >= base_row) & (src < base_row + ROWS_PER_TILE)
        local_r = jnp.where(mask, src - base_row, 0)
        plsc.store_scatter(a_v, [local_r, dst], ones, mask=mask)
    pltpu.sync_copy(a_v, a_hbm.at[pl.ds(base_row, ROWS_PER_TILE), :])


def _bfs_body(a_ref, dist_ref, abf_ref, frontier_ref, cnt_ref):
    abf_ref[...] = a_ref[...].astype(jnp.bfloat16)
    row = lax.broadcasted_iota(jnp.int32, (N, N), 0)
    col = lax.broadcasted_iota(jnp.int32, (N, N), 1)
    diag = row == col
    frontier_ref[...] = jnp.where(diag, 1.0, 0.0).astype(jnp.bfloat16)
    dist_ref[...] = jnp.zeros((N, N), jnp.int32)

    def body(i, done):
        @pl.when(jnp.logical_not(done))
        def _():
            nxt = lax.dot_general(
                frontier_ref[...], abf_ref[...],
                (((1,), (0,)), ((), ())),
                preferred_element_type=jnp.float32,
            )
            new = (nxt > 0.0) & (dist_ref[...] == 0) & jnp.logical_not(diag)
            dist_ref[...] = jnp.where(new, i + 1, dist_ref[...])
            frontier_ref[...] = new.astype(jnp.bfloat16)
            cnt_ref[0] = jnp.sum(new.astype(jnp.int32))

        return jnp.logical_or(done, cnt_ref[0] == 0)

    lax.fori_loop(0, MAX_ITERS, body, False)


_bfs = pl.pallas_call(
    _bfs_body,
    out_shape=jax.ShapeDtypeStruct((N, N), jnp.int32),
    scratch_shapes=[
        pltpu.VMEM((N, N), jnp.bfloat16),
        pltpu.VMEM((N, N), jnp.bfloat16),
        pltpu.SMEM((1,), jnp.int32),
    ],
)


@functools.partial(
    pl.kernel,
    out_type=jax.ShapeDtypeStruct((HEADS, N, N), jnp.float32),
    mesh=_mesh,
    scratch_types=[
        pltpu.VMEM((HEADS * TABLE_V,), jnp.float32),
        pltpu.VMEM((2, ROWS_PER_CHUNK, N), jnp.int32),
        pltpu.VMEM((2, HEADS, ROWS_PER_CHUNK, N), jnp.float32),
        pltpu.SemaphoreType.DMA((2,)),
        pltpu.SemaphoreType.DMA((2,)),
    ],
    compiler_params=_sc_params,
)
def _lookup(dist_hbm, ftab_hbm, out_hbm, tab_v, idx_v, out_v, in_sems, out_sems):
    wid = lax.axis_index("s") * NC + lax.axis_index("c")
    row_base = wid * ROWS_PER_TILE
    pltpu.sync_copy(ftab_hbm, tab_v)

    def in_desc(c, b):
        return pltpu.make_async_copy(
            dist_hbm.at[pl.ds(row_base + c * ROWS_PER_CHUNK, ROWS_PER_CHUNK), :],
            idx_v.at[b],
            in_sems.at[b],
        )

    def out_desc(c, b):
        return pltpu.make_async_copy(
            out_v.at[b],
            out_hbm.at[:, pl.ds(row_base + c * ROWS_PER_CHUNK, ROWS_PER_CHUNK), :],
            out_sems.at[b],
        )

    in_desc(0, 0).start()
    in_desc(1, 1).start()

    def chunk_pair(cp, carry):
        for b in range(2):
            c = cp * 2 + b
            in_desc(c, b).wait()

            @pl.when(cp > 0)
            def _():
                out_desc(c, b).wait()

            for r in range(ROWS_PER_CHUNK):
                @plsc.parallel_loop(0, N // LANES, 1, unroll=4)
                def _(cg):
                    col = cg * LANES
                    iv = idx_v[b, r, pl.ds(col, LANES)]
                    for h in range(HEADS):
                        vals = plsc.load_gather(tab_v, [iv + h * TABLE_V])
                        out_v[b, h, r, pl.ds(col, LANES)] = vals

            out_desc(c, b).start()

            @pl.when(c + 2 < CHUNKS)
            def _():
                in_desc(c + 2, b).start()
        return carry

    lax.fori_loop(0, CHUNKS // 2, chunk_pair, 0)
    out_desc(CHUNKS - 2, 0).wait()
    out_desc(CHUNKS - 1, 1).wait()


def kernel(x, edge_idx, table):
    del x
    a = _build_adj(edge_idx)
    dist = _bfs(a)
    ftab = jnp.transpose(table).reshape(HEADS * TABLE_V)
    return _lookup(dist, ftab)


# speedup vs baseline: 1.4945x; 1.0048x over previous
import functools

import jax
import jax.numpy as jnp
from jax import lax
from jax.experimental import pallas as pl
from jax.experimental.pallas import tpu as pltpu
from jax.experimental.pallas import tpu_sc as plsc

N = 1024
E = 16384
HEADS = 16
TABLE_V = 50
MAX_ITERS = 49

NC = 2
NS = 16
LANES = 16
NW = NC * NS

ROWS_PER_TILE = N // NW
CHUNK = 2048
ROWS_PER_CHUNK = CHUNK // N
CHUNKS = ROWS_PER_TILE // ROWS_PER_CHUNK

_mesh = plsc.VectorSubcoreMesh(core_axis_name="c", subcore_axis_name="s")
_sc_params = pltpu.CompilerParams(needs_layout_passes=False)


@functools.partial(
    pl.kernel,
    out_type=jax.ShapeDtypeStruct((N, N), jnp.float32),
    mesh=_mesh,
    scratch_types=[
        pltpu.VMEM((2, E), jnp.int32),
        pltpu.VMEM((ROWS_PER_TILE, N), jnp.float32),
    ],
    compiler_params=_sc_params,
)
def _build_adj(edge_hbm, a_hbm, edges_v, a_v):
    wid = lax.axis_index("s") * NC + lax.axis_index("c")
    base_row = wid * ROWS_PER_TILE
    pltpu.sync_copy(edge_hbm, edges_v)

    @plsc.parallel_loop(0, ROWS_PER_TILE * N // LANES, 1, unroll=4)
    def _(i):
        r = lax.shift_right_logical(i, 6)
        c = lax.shift_left(jnp.bitwise_and(i, 63), 4)
        a_v[r, pl.ds(c, LANES)] = jnp.zeros((LANES,), jnp.float32)

    ones = jnp.ones((LANES,), jnp.float32)

    @plsc.parallel_loop(0, E // LANES, 1, unroll=4)
    def _(i):
        src = edges_v[0, pl.ds(i * LANES, LANES)]
        dst = edges_v[1, pl.ds(i * LANES, LANES)]
        mask = (src >= base_row) & (src < base_row + ROWS_PER_TILE)
        local_r = jnp.where(mask, src - base_row, 0)
        plsc.store_scatter(a_v, [local_r, dst], ones, mask=mask)
    pltpu.sync_copy(a_v, a_hbm.at[pl.ds(base_row, ROWS_PER_TILE), :])


def _bfs_body(a_ref, dist_ref, abf_ref, frontier_ref, cnt_ref):
    abf_ref[...] = a_ref[...].astype(jnp.int8)
    row = lax.broadcasted_iota(jnp.int32, (N, N), 0)
    col = lax.broadcasted_iota(jnp.int32, (N, N), 1)
    diag = row == col
    frontier_ref[...] = jnp.where(diag, 1, 0).astype(jnp.int8)
    dist_ref[...] = jnp.zeros((N, N), jnp.int32)

    def body(i, done):
        @pl.when(jnp.logical_not(done))
        def _():
            nxt = lax.dot_general(
                frontier_ref[...], abf_ref[...],
                (((1,), (0,)), ((), ())),
                preferred_element_type=jnp.int32,
            )
            new = (nxt > 0) & (dist_ref[...] == 0) & jnp.logical_not(diag)
            dist_ref[...] = jnp.where(new, i + 1, dist_ref[...])
            frontier_ref[...] = new.astype(jnp.int8)
            cnt_ref[0] = jnp.sum(new.astype(jnp.int32))

        return jnp.logical_or(done, cnt_ref[0] == 0)

    lax.fori_loop(0, MAX_ITERS, body, False)


_bfs = pl.pallas_call(
    _bfs_body,
    out_shape=jax.ShapeDtypeStruct((N, N), jnp.int32),
    scratch_shapes=[
        pltpu.VMEM((N, N), jnp.int8),
        pltpu.VMEM((N, N), jnp.int8),
        pltpu.SMEM((1,), jnp.int32),
    ],
)


@functools.partial(
    pl.kernel,
    out_type=jax.ShapeDtypeStruct((HEADS, N, N), jnp.float32),
    mesh=_mesh,
    scratch_types=[
        pltpu.VMEM((HEADS * TABLE_V,), jnp.float32),
        pltpu.VMEM((2, ROWS_PER_CHUNK, N), jnp.int32),
        pltpu.VMEM((2, HEADS, ROWS_PER_CHUNK, N), jnp.float32),
        pltpu.SemaphoreType.DMA((2,)),
        pltpu.SemaphoreType.DMA((2,)),
    ],
    compiler_params=_sc_params,
)
def _lookup(dist_hbm, ftab_hbm, out_hbm, tab_v, idx_v, out_v, in_sems, out_sems):
    wid = lax.axis_index("s") * NC + lax.axis_index("c")
    row_base = wid * ROWS_PER_TILE
    pltpu.sync_copy(ftab_hbm, tab_v)

    def in_desc(c, b):
        return pltpu.make_async_copy(
            dist_hbm.at[pl.ds(row_base + c * ROWS_PER_CHUNK, ROWS_PER_CHUNK), :],
            idx_v.at[b],
            in_sems.at[b],
        )

    def out_desc(c, b):
        return pltpu.make_async_copy(
            out_v.at[b],
            out_hbm.at[:, pl.ds(row_base + c * ROWS_PER_CHUNK, ROWS_PER_CHUNK), :],
            out_sems.at[b],
        )

    in_desc(0, 0).start()
    in_desc(1, 1).start()

    def chunk_pair(cp, carry):
        for b in range(2):
            c = cp * 2 + b
            in_desc(c, b).wait()

            @pl.when(cp > 0)
            def _():
                out_desc(c, b).wait()

            for r in range(ROWS_PER_CHUNK):
                @plsc.parallel_loop(0, N // LANES, 1, unroll=4)
                def _(cg):
                    col = cg * LANES
                    iv = idx_v[b, r, pl.ds(col, LANES)]
                    for h in range(HEADS):
                        vals = plsc.load_gather(tab_v, [iv + h * TABLE_V])
                        out_v[b, h, r, pl.ds(col, LANES)] = vals

            out_desc(c, b).start()

            @pl.when(c + 2 < CHUNKS)
            def _():
                in_desc(c + 2, b).start()
        return carry

    lax.fori_loop(0, CHUNKS // 2, chunk_pair, 0)
    out_desc(CHUNKS - 2, 0).wait()
    out_desc(CHUNKS - 1, 1).wait()


def kernel(x, edge_idx, table):
    del x
    a = _build_adj(edge_idx)
    dist = _bfs(a)
    ftab = jnp.transpose(table).reshape(HEADS * TABLE_V)
    return _lookup(dist, ftab)
